# baseline (speedup 1.0000x reference)
import functools

import jax
import jax.numpy as jnp
from jax import lax
from jax.experimental import pallas as pl
from jax.experimental.pallas import tpu as pltpu
from jax.experimental.pallas import tpu_sc as plsc

_NUM_CORES = 2
_NUM_SUBCORES = 16
_NW = _NUM_CORES * _NUM_SUBCORES
_CHUNK = 128


def _sc_gather_user(uidx4, userEmb4):
    _, nch, _ = uidx4.shape
    B = _NW * nch * _CHUNK
    EMB4 = userEmb4.shape[1]
    bpw = B // _NW

    mesh = plsc.VectorSubcoreMesh(core_axis_name="c", subcore_axis_name="s")

    @functools.partial(
        pl.kernel,
        mesh=mesh,
        compiler_params=pltpu.CompilerParams(use_tc_tiling_on_sc=True),
        out_type=jax.ShapeDtypeStruct((B, EMB4), jnp.float32),
        scratch_types=[
            pltpu.VMEM((nch, _CHUNK), jnp.int32),
            pltpu.VMEM((2, _CHUNK, 128), jnp.float32),
            pltpu.SemaphoreType.DMA,
            pltpu.SemaphoreType.DMA,
        ],
    )
    def k(uemb_hbm, uidx_hbm, ou, uidx_v, urows2, sem_g, sem_o):
        wid = lax.axis_index("s") * _NUM_CORES + lax.axis_index("c")
        base = wid * bpw
        pltpu.sync_copy(uidx_hbm.at[wid], uidx_v)
        out_pend = [None, None]
        for j in range(nch):
            b = j & 1
            if out_pend[b] is not None:
                out_pend[b].wait()
            pltpu.async_copy(uemb_hbm.at[uidx_v.at[j]], urows2.at[b], sem_g).wait()
            osl = pl.ds(base + j * _CHUNK, _CHUNK)
            out_pend[b] = pltpu.async_copy(urows2.at[b], ou.at[osl], sem_o)
        for pend in out_pend:
            if pend is not None:
                pend.wait()

    return k(userEmb4, uidx4)


def _sc_gather_movie_bias(uidx, midx, movieEmb, ubflat, mbflat):
    _, nch, _ = uidx.shape
    B = _NW * nch * _CHUNK
    EMB = movieEmb.shape[1]
    bpw = B // _NW

    mesh = plsc.VectorSubcoreMesh(core_axis_name="c", subcore_axis_name="s")

    @functools.partial(
        pl.kernel,
        mesh=mesh,
        compiler_params=pltpu.CompilerParams(use_tc_tiling_on_sc=False),
        out_type=(
            jax.ShapeDtypeStruct((B, EMB), jnp.float32),
            jax.ShapeDtypeStruct((B,), jnp.float32),
            jax.ShapeDtypeStruct((B,), jnp.float32),
        ),
        scratch_types=[
            pltpu.VMEM((nch, _CHUNK), jnp.int32),
            pltpu.VMEM((nch, _CHUNK), jnp.int32),
            pltpu.VMEM((bpw, EMB), jnp.float32),
            pltpu.VMEM((bpw,), jnp.float32),
            pltpu.VMEM((bpw,), jnp.float32),
            pltpu.SemaphoreType.DMA,
        ],
    )
    def k(memb_hbm, ub_hbm, mb_hbm, uidx_hbm, midx_hbm,
          om, oub, omb, uidx_v, midx_v, mrows, ub_v, mb_v, sem):
        wid = lax.axis_index("s") * _NUM_CORES + lax.axis_index("c")
        base = wid * bpw
        pltpu.sync_copy(uidx_hbm.at[wid], uidx_v)
        pltpu.sync_copy(midx_hbm.at[wid], midx_v)
        copies = []
        for j in range(nch):
            sl = pl.ds(j * _CHUNK, _CHUNK)
            copies.append(pltpu.async_copy(memb_hbm.at[midx_v.at[j]], mrows.at[sl], sem))
            copies.append(pltpu.async_copy(ub_hbm.at[uidx_v.at[j]], ub_v.at[sl], sem))
            copies.append(pltpu.async_copy(mb_hbm.at[midx_v.at[j]], mb_v.at[sl], sem))
        for c in copies:
            c.wait()
        osl = pl.ds(base, bpw)
        pltpu.sync_copy(mrows, om.at[osl])
        pltpu.sync_copy(ub_v, oub.at[osl])
        pltpu.sync_copy(mb_v, omb.at[osl])

    return k(movieEmb, ubflat, mbflat, uidx, midx)


def _extract32(rows, selcol):
    out = jnp.where(selcol == 0, rows[:, 0:32], 0.0)
    out = out + jnp.where(selcol == 1, rows[:, 32:64], 0.0)
    out = out + jnp.where(selcol == 2, rows[:, 64:96], 0.0)
    out = out + jnp.where(selcol == 3, rows[:, 96:128], 0.0)
    return out


def _tc_body(const_ref, u_ref, m_ref, goh_ref, ub_ref, mb_ref,
             usel_ref,
             gwt_ref, gb_ref, w1u_ref, w1m_ref, w1g_ref, b1_ref, w2_ref,
             o_ref):
    u = _extract32(u_ref[...], usel_ref[...][:, None])
    m = m_ref[...]
    g = jnp.dot(goh_ref[...], gwt_ref[...], preferred_element_type=jnp.float32)
    g = jnp.maximum(g + gb_ref[...], 0.0)
    h = jnp.dot(u, w1u_ref[...], preferred_element_type=jnp.float32)
    h = h + jnp.dot(m, w1m_ref[...], preferred_element_type=jnp.float32)
    h = h + jnp.dot(g, w1g_ref[...], preferred_element_type=jnp.float32)
    h = jnp.maximum(h + b1_ref[...], 0.0)
    mlp = jnp.sum(h * w2_ref[...], axis=1)
    dot = jnp.sum(u * m, axis=1)
    o_ref[...] = dot + mlp + ub_ref[...] + mb_ref[...] + const_ref[0]


def _tc_dense(urows, m, genreOH, ub, mb, usel,
              gW, gb, w1, b1, w2, const):
    B = urows.shape[0]
    NG = genreOH.shape[1]
    GE = gW.shape[0]
    HL = w1.shape[0]
    EMB = (w1.shape[1] - GE) // 2
    BLK = 2048
    grid = (B // BLK,)

    gWt = gW.T
    w1t = w1.T
    w1u = w1t[:EMB]
    w1m = w1t[EMB:2 * EMB]
    w1g = w1t[2 * EMB:]
    w2row = w2[0]

    full = lambda shape: pl.BlockSpec(shape, lambda i: (0,) * len(shape))
    return pl.pallas_call(
        _tc_body,
        grid=grid,
        in_specs=[
            pl.BlockSpec(memory_space=pltpu.SMEM),
            pl.BlockSpec((BLK, 128), lambda i: (i, 0)),
            pl.BlockSpec((BLK, EMB), lambda i: (i, 0)),
            pl.BlockSpec((BLK, NG), lambda i: (i, 0)),
            pl.BlockSpec((BLK,), lambda i: (i,)),
            pl.BlockSpec((BLK,), lambda i: (i,)),
            pl.BlockSpec((BLK,), lambda i: (i,)),
            full((NG, GE)),
            full((GE,)),
            full((EMB, HL)),
            full((EMB, HL)),
            full((GE, HL)),
            full((HL,)),
            full((HL,)),
        ],
        out_specs=pl.BlockSpec((BLK,), lambda i: (i,)),
        out_shape=jax.ShapeDtypeStruct((B,), jnp.float32),
    )(const, urows, m, genreOH, ub, mb, usel,
      gWt, gb, w1u, w1m, w1g, b1, w2row)


def kernel(userOH, moveOH, genreOH, userEmb, movieEmb, userBiasT, movieBiasT,
           bias, gW, gb, w1, b1, w2, b2):
    B = userOH.shape[0]
    nch = B // (_NW * _CHUNK)
    uoh = userOH.astype(jnp.int32)
    moh = moveOH.astype(jnp.int32)
    uidx4 = (uoh >> 2).reshape(_NW, nch, _CHUNK)
    uidx = uoh.reshape(_NW, nch, _CHUNK)
    midx = moh.reshape(_NW, nch, _CHUNK)
    userEmb4 = userEmb.reshape(userEmb.shape[0] // 4, 128)
    ubflat = userBiasT.reshape(-1)
    mbflat = movieBiasT.reshape(-1)

    urows = _sc_gather_user(uidx4, userEmb4)
    m, ub, mb = _sc_gather_movie_bias(uidx, midx, movieEmb, ubflat, mbflat)
    const = (bias + b2).reshape(1)
    return _tc_dense(urows, m, genreOH, ub, mb, uoh & 3,
                     gW, gb, w1, b1, w2, const)


# speedup vs baseline: 1.0406x; 1.0406x over previous
import functools

import jax
import jax.numpy as jnp
from jax import lax
from jax.experimental import pallas as pl
from jax.experimental.pallas import tpu as pltpu
from jax.experimental.pallas import tpu_sc as plsc

_NUM_CORES = 2
_NUM_SUBCORES = 16
_NW = _NUM_CORES * _NUM_SUBCORES
_CHUNK = 128


def _sc_gather_user(uidx4, userEmb4):
    _, nch, _ = uidx4.shape
    B = _NW * nch * _CHUNK
    EMB4 = userEmb4.shape[1]
    bpw = B // _NW

    mesh = plsc.VectorSubcoreMesh(core_axis_name="c", subcore_axis_name="s")

    @functools.partial(
        pl.kernel,
        mesh=mesh,
        compiler_params=pltpu.CompilerParams(use_tc_tiling_on_sc=True),
        out_type=jax.ShapeDtypeStruct((B, EMB4), jnp.float32),
        scratch_types=[
            pltpu.VMEM((nch, _CHUNK), jnp.int32),
            pltpu.VMEM((2, _CHUNK, 128), jnp.float32),
            pltpu.SemaphoreType.DMA,
            pltpu.SemaphoreType.DMA,
        ],
    )
    def k(uemb_hbm, uidx_hbm, ou, uidx_v, urows2, sem_g, sem_o):
        wid = lax.axis_index("s") * _NUM_CORES + lax.axis_index("c")
        base = wid * bpw
        pltpu.sync_copy(uidx_hbm.at[wid], uidx_v)
        out_pend = [None, None]
        for j in range(nch):
            b = j & 1
            if out_pend[b] is not None:
                out_pend[b].wait()
            pltpu.async_copy(uemb_hbm.at[uidx_v.at[j]], urows2.at[b], sem_g).wait()
            osl = pl.ds(base + j * _CHUNK, _CHUNK)
            out_pend[b] = pltpu.async_copy(urows2.at[b], ou.at[osl], sem_o)
        for pend in out_pend:
            if pend is not None:
                pend.wait()

    return k(userEmb4, uidx4)


def _sc_gather_movie_bias(uidx, midx, movieEmb, ubflat, mbflat):
    _, nch, _ = uidx.shape
    B = _NW * nch * _CHUNK
    EMB = movieEmb.shape[1]
    bpw = B // _NW

    mesh = plsc.VectorSubcoreMesh(core_axis_name="c", subcore_axis_name="s")

    @functools.partial(
        pl.kernel,
        mesh=mesh,
        compiler_params=pltpu.CompilerParams(use_tc_tiling_on_sc=False),
        out_type=(
            jax.ShapeDtypeStruct((B, EMB), jnp.float32),
            jax.ShapeDtypeStruct((B,), jnp.float32),
            jax.ShapeDtypeStruct((B,), jnp.float32),
        ),
        scratch_types=[
            pltpu.VMEM((nch, _CHUNK), jnp.int32),
            pltpu.VMEM((nch, _CHUNK), jnp.int32),
            pltpu.VMEM((bpw, EMB), jnp.float32),
            pltpu.VMEM((bpw,), jnp.float32),
            pltpu.VMEM((bpw,), jnp.float32),
            pltpu.SemaphoreType.DMA,
        ],
    )
    def k(memb_hbm, ub_hbm, mb_hbm, uidx_hbm, midx_hbm,
          om, oub, omb, uidx_v, midx_v, mrows, ub_v, mb_v, sem):
        wid = lax.axis_index("s") * _NUM_CORES + lax.axis_index("c")
        base = wid * bpw
        pltpu.sync_copy(uidx_hbm.at[wid], uidx_v)
        pltpu.sync_copy(midx_hbm.at[wid], midx_v)
        copies = []
        for j in range(nch):
            sl = pl.ds(j * _CHUNK, _CHUNK)
            copies.append(pltpu.async_copy(memb_hbm.at[midx_v.at[j]], mrows.at[sl], sem))
            copies.append(pltpu.async_copy(ub_hbm.at[uidx_v.at[j]], ub_v.at[sl], sem))
            copies.append(pltpu.async_copy(mb_hbm.at[midx_v.at[j]], mb_v.at[sl], sem))
        for c in copies:
            c.wait()
        osl = pl.ds(base, bpw)
        pltpu.sync_copy(mrows, om.at[osl])
        pltpu.sync_copy(ub_v, oub.at[osl])
        pltpu.sync_copy(mb_v, omb.at[osl])

    return k(movieEmb, ubflat, mbflat, uidx, midx)


def _extract32(rows, selcol):
    out = jnp.where(selcol == 0, rows[:, 0:32], 0.0)
    out = out + jnp.where(selcol == 1, rows[:, 32:64], 0.0)
    out = out + jnp.where(selcol == 2, rows[:, 64:96], 0.0)
    out = out + jnp.where(selcol == 3, rows[:, 96:128], 0.0)
    return out


def _tc_body(const_ref, u_ref, m_ref, goh_ref, ub_ref, mb_ref,
             usel_ref,
             gwt_ref, gb_ref, w1u_ref, w1m_ref, w1g_ref, b1_ref, w2_ref,
             o_ref):
    u = u_ref[...][:, :32]
    m = m_ref[...]
    g = jnp.dot(goh_ref[...], gwt_ref[...], preferred_element_type=jnp.float32)
    g = jnp.maximum(g + gb_ref[...], 0.0)
    h = jnp.dot(u, w1u_ref[...], preferred_element_type=jnp.float32)
    h = h + jnp.dot(m, w1m_ref[...], preferred_element_type=jnp.float32)
    h = h + jnp.dot(g, w1g_ref[...], preferred_element_type=jnp.float32)
    h = jnp.maximum(h + b1_ref[...], 0.0)
    mlp = jnp.sum(h * w2_ref[...], axis=1)
    dot = jnp.sum(u * m, axis=1)
    o_ref[...] = dot + mlp + ub_ref[...] + mb_ref[...] + const_ref[0]


def _tc_dense(urows, m, genreOH, ub, mb, usel,
              gW, gb, w1, b1, w2, const):
    B = urows.shape[0]
    NG = genreOH.shape[1]
    GE = gW.shape[0]
    HL = w1.shape[0]
    EMB = (w1.shape[1] - GE) // 2
    BLK = 2048
    grid = (B // BLK,)

    gWt = gW.T
    w1t = w1.T
    w1u = w1t[:EMB]
    w1m = w1t[EMB:2 * EMB]
    w1g = w1t[2 * EMB:]
    w2row = w2[0]

    full = lambda shape: pl.BlockSpec(shape, lambda i: (0,) * len(shape))
    return pl.pallas_call(
        _tc_body,
        grid=grid,
        in_specs=[
            pl.BlockSpec(memory_space=pltpu.SMEM),
            pl.BlockSpec((BLK, 128), lambda i: (i, 0)),
            pl.BlockSpec((BLK, EMB), lambda i: (i, 0)),
            pl.BlockSpec((BLK, NG), lambda i: (i, 0)),
            pl.BlockSpec((BLK,), lambda i: (i,)),
            pl.BlockSpec((BLK,), lambda i: (i,)),
            pl.BlockSpec((BLK,), lambda i: (i,)),
            full((NG, GE)),
            full((GE,)),
            full((EMB, HL)),
            full((EMB, HL)),
            full((GE, HL)),
            full((HL,)),
            full((HL,)),
        ],
        out_specs=pl.BlockSpec((BLK,), lambda i: (i,)),
        out_shape=jax.ShapeDtypeStruct((B,), jnp.float32),
    )(const, urows, m, genreOH, ub, mb, usel,
      gWt, gb, w1u, w1m, w1g, b1, w2row)


def kernel(userOH, moveOH, genreOH, userEmb, movieEmb, userBiasT, movieBiasT,
           bias, gW, gb, w1, b1, w2, b2):
    B = userOH.shape[0]
    nch = B // (_NW * _CHUNK)
    uoh = userOH.astype(jnp.int32)
    moh = moveOH.astype(jnp.int32)
    uidx = uoh.reshape(_NW, nch, _CHUNK)
    midx = moh.reshape(_NW, nch, _CHUNK)
    userEmbPad = jnp.pad(userEmb, ((0, 0), (0, 96)))
    ubflat = userBiasT.reshape(-1)
    mbflat = movieBiasT.reshape(-1)

    urows = _sc_gather_user(uidx, userEmbPad)
    m, ub, mb = _sc_gather_movie_bias(uidx, midx, movieEmb, ubflat, mbflat)
    const = (bias + b2).reshape(1)
    return _tc_dense(urows, m, genreOH, ub, mb, uoh & 3,
                     gW, gb, w1, b1, w2, const)
